# baseline (speedup 1.0000x reference)
import functools

import jax
import jax.numpy as jnp
from jax import lax
from jax.experimental import pallas as pl
from jax.experimental.pallas import tpu as pltpu
from jax.experimental.pallas import tpu_sc as plsc

KNN = 16
ROW_BLOCK = 400
EDGE_BLOCK = 400


def _bn_cols(x, g, b, eps=1e-5):
    m = jnp.mean(x, axis=0)
    v = jnp.var(x, axis=0)
    return g * (x - m) / jnp.sqrt(v + eps) + b


def _mm(a, b):
    return lax.dot_general(a, b, (((1,), (0,)), ((), ())),
                           preferred_element_type=jnp.float32)


def _mm_t(a, b):
    return lax.dot_general(a, b, (((1,), (1,)), ((), ())),
                           preferred_element_type=jnp.float32)


def _leaky(z):
    return jnp.where(z >= 0, z, 0.01 * z)


def _leaky2(z):
    return jnp.where(z >= 0, z, 1e-4 * z)


def _knn_body(hq_ref, hf_ref, idx_ref):
    xc = hq_ref[...]
    hf = hf_ref[...]
    n = hf.shape[0]
    r = xc.shape[0]
    sqc = jnp.sum(xc * xc, axis=1, keepdims=True)
    ones = jnp.ones((1, hf.shape[1]), jnp.float32)
    sqr = lax.dot_general(ones, hf * hf, (((1,), (1,)), ((), ())),
                          preferred_element_type=jnp.float32,
                          precision=lax.Precision.HIGHEST)
    d = (sqc + sqr) - 2.0 * _mm_t(xc, hf)
    iota = lax.broadcasted_iota(jnp.int32, (r, n), 1)
    cols = []
    big = jnp.int32(2**30)
    for _ in range(KNN):
        m = jnp.min(d, axis=1, keepdims=True)
        cand = jnp.where(d == m, iota, big)
        sel = jnp.min(cand, axis=1, keepdims=True)
        cols.append(sel)
        d = jnp.where(iota == sel, jnp.inf, d)
    idx_ref[...] = jnp.concatenate(cols, axis=1)


def _knn(h):
    n, hdim = h.shape
    rb = ROW_BLOCK if n % ROW_BLOCK == 0 else n
    return pl.pallas_call(
        _knn_body,
        grid=(n // rb,),
        in_specs=[
            pl.BlockSpec((rb, hdim), lambda i: (i, 0)),
            pl.BlockSpec((n, hdim), lambda i: (0, 0)),
        ],
        out_specs=pl.BlockSpec((rb, KNN), lambda i: (i, 0)),
        out_shape=jax.ShapeDtypeStruct((n, KNN), jnp.int32),
    )(h, h)


_SC_INFO = plsc.get_sparse_core_info()
_NW = _SC_INFO.num_cores * _SC_INFO.num_subcores


def _pad_rows(idx, nw):
    n = idx.shape[0]
    npad = -n % (8 * nw)
    if npad:
        idx = jnp.concatenate([idx, jnp.zeros((npad,) + idx.shape[1:], idx.dtype)], 0)
    return idx, n + npad


def _gather_rows(table, idx):
    n, k = idx.shape
    d = table.shape[1]
    idx_p, n_p = _pad_rows(idx, _NW)
    per_w = (n_p * k) // _NW
    steps = per_w // k
    mesh = plsc.VectorSubcoreMesh(core_axis_name="c", subcore_axis_name="s")

    @functools.partial(
        pl.kernel, mesh=mesh,
        out_type=jax.ShapeDtypeStruct((n_p * k, d), jnp.float32),
        scratch_types=[
            pltpu.VMEM((per_w,), jnp.int32),
            pltpu.VMEM((k, d), jnp.float32),
            pltpu.VMEM((k, d), jnp.float32),
            pltpu.SemaphoreType.DMA,
            pltpu.SemaphoreType.DMA,
        ],
    )
    def gk(idx_hbm, tab_hbm, out_hbm, idx_v, buf0, buf1, sem0, sem1):
        wid = lax.axis_index("s") * _SC_INFO.num_cores + lax.axis_index("c")
        base = wid * per_w
        pltpu.sync_copy(idx_hbm.at[pl.ds(base, per_w)], idx_v)

        def body(step, _):
            pltpu.async_copy(tab_hbm.at[idx_v[pl.ds(step * k, k)]], buf0, sem0).wait()
            pltpu.sync_copy(buf0, out_hbm.at[pl.ds(base + step * k, k)])
            return 0

        lax.fori_loop(0, steps, body, 0)

    return gk(idx_p.reshape(-1), table)


def _gather_max(table, idx):
    n, k = idx.shape
    d = table.shape[1]
    lanes = _SC_INFO.num_lanes
    idx_p, n_p = _pad_rows(idx, _NW)
    rows_w = n_p // _NW
    mesh = plsc.VectorSubcoreMesh(core_axis_name="c", subcore_axis_name="s")

    @functools.partial(
        pl.kernel, mesh=mesh,
        out_type=jax.ShapeDtypeStruct((n_p, d), jnp.float32),
        scratch_types=[
            pltpu.VMEM((rows_w * k,), jnp.int32),
            pltpu.VMEM((k, d), jnp.float32),
            pltpu.VMEM((rows_w, d), jnp.float32),
            pltpu.SemaphoreType.DMA,
        ],
    )
    def gmk(idx_hbm, tab_hbm, out_hbm, idx_v, buf, acc_v, sem):
        wid = lax.axis_index("s") * _SC_INFO.num_cores + lax.axis_index("c")
        base = wid * rows_w
        pltpu.sync_copy(idx_hbm.at[pl.ds(base * k, rows_w * k)], idx_v)

        def body(row, _):
            pltpu.async_copy(tab_hbm.at[idx_v[pl.ds(row * k, k)]], buf, sem).wait()
            for c in range(d // lanes):
                acc = buf[0, pl.ds(c * lanes, lanes)]
                for r in range(1, k):
                    acc = jnp.maximum(acc, buf[r, pl.ds(c * lanes, lanes)])
                acc_v[row, pl.ds(c * lanes, lanes)] = acc
            return 0

        lax.fori_loop(0, rows_w, body, 0)
        pltpu.sync_copy(acc_v, out_hbm.at[pl.ds(base, rows_w)])

    out = gmk(idx_p.reshape(-1), table)
    return out[:n]


def _edge_body(hq_ref, xj_ref, at_ref, wb_ref, h1_ref):
    r, dd = at_ref.shape
    xi = jnp.broadcast_to(hq_ref[...][:, None, :], (r, KNN, dd))
    diff = (xj_ref[...].reshape(r, KNN, dd) - xi).astype(jnp.bfloat16)
    p = lax.dot_general(diff.reshape(r * KNN, dd), wb_ref[...],
                        (((1,), (0,)), ((), ())),
                        preferred_element_type=jnp.float32)
    z = _leaky(at_ref[...][:, None, :] + p.reshape(r, KNN, dd))
    h1_ref[...] = _leaky(jnp.max(z, axis=1))


def _edge_layer1(h, at, xj_flat, Wb):
    n, dd = h.shape
    rb = EDGE_BLOCK if n % EDGE_BLOCK == 0 else n
    return pl.pallas_call(
        _edge_body,
        grid=(n // rb,),
        in_specs=[
            pl.BlockSpec((rb, dd), lambda i: (i, 0)),
            pl.BlockSpec((rb * KNN, dd), lambda i: (i, 0)),
            pl.BlockSpec((rb, dd), lambda i: (i, 0)),
            pl.BlockSpec((dd, dd), lambda i: (0, 0)),
        ],
        out_specs=pl.BlockSpec((rb, dd), lambda i: (i, 0)),
        out_shape=jax.ShapeDtypeStruct((n, dd), jnp.float32),
    )(h, xj_flat, at, Wb)


def _at_proj(h, Wt, b):
    n, dd = h.shape
    def body(h_ref, wt_ref, b_ref, out_ref):
        hb = h_ref[...].astype(jnp.bfloat16)
        out_ref[...] = lax.dot_general(
            hb, wt_ref[...], (((1,), (0,)), ((), ())),
            preferred_element_type=jnp.float32) + b_ref[...]
    return pl.pallas_call(
        body, out_shape=jax.ShapeDtypeStruct((n, dd), jnp.float32),
    )(h, Wt.astype(jnp.bfloat16), b.reshape(1, -1))


def _ab_proj(h1, eA, eB, eb):
    n, dd = h1.shape
    def body(h_ref, ea_ref, ebm_ref, eb_ref, a_ref, b_ref):
        h = h_ref[...]
        a_ref[...] = _mm(h, ea_ref[...]) + eb_ref[...]
        b_ref[...] = _mm(h, ebm_ref[...])
    return pl.pallas_call(
        body, out_shape=[jax.ShapeDtypeStruct((n, dd), jnp.float32)] * 2,
    )(h1, eA, eB, eb.reshape(1, -1))


def _head_body(a_ref, m_ref, lng_ref, lnb_ref, w_ref, b_ref, out_ref):
    h = _leaky2(a_ref[...] + m_ref[...])
    mu = jnp.mean(h, axis=1, keepdims=True)
    var = jnp.mean((h - mu) ** 2, axis=1, keepdims=True)
    hn = lng_ref[...] * (h - mu) / jnp.sqrt(var + 1e-5) + lnb_ref[...]
    out_ref[...] = _mm(hn, w_ref[...]) + b_ref[...]


def _heads(a1, m1, ln_g, ln_b, Wh, bh):
    n, _ = a1.shape
    cols = Wh.shape[1]
    return pl.pallas_call(
        _head_body,
        out_shape=jax.ShapeDtypeStruct((n, cols), jnp.float32),
    )(a1, m1, ln_g.reshape(1, -1), ln_b.reshape(1, -1), Wh, bh.reshape(1, -1))


def kernel(x, bn_g, bn_b, W1, b1, g1, be1, W2, b2, g2, be2, eW0, eb0, eW1,
           eb1, ln_g, ln_b, Wc, bc, Wn, bn2, Wcat0, bcat0, Wcat1, bcat1):
    hdim = W1.shape[1]

    h = _bn_cols(x, bn_g, bn_b)
    h = h @ W1 + b1
    h = jax.nn.relu(_bn_cols(h, g1, be1))
    h = h @ W2 + b2
    h = _bn_cols(h, g2, be2)

    idx0 = _knn(h)
    at0 = _at_proj(h, eW0[:hdim], eb0)
    xj = _gather_rows(h, idx0)
    h1 = _edge_layer1(h, at0, xj, eW0[hdim:].astype(jnp.bfloat16))

    idx1 = _knn(h1)
    eA1, eB1 = eW1[:hdim] - eW1[hdim:], eW1[hdim:]
    a1, b1m = _ab_proj(h1, eA1, eB1, eb1)
    m1 = _gather_max(b1m, idx1)

    Wh = jnp.concatenate([Wc, Wn, Wcat0, Wcat1], axis=1)
    bh = jnp.concatenate([bc, bn2, bcat0, bcat1], axis=0)
    out = _heads(a1, m1, ln_g, ln_b, Wh, bh)
    n_cls, n_num = Wc.shape[1], Wn.shape[1]
    c0, c1 = Wcat0.shape[1], Wcat1.shape[1]
    logits = out[:, :n_cls]
    num_rec = out[:, n_cls:n_cls + n_num]
    cat0 = out[:, n_cls + n_num:n_cls + n_num + c0]
    cat1 = out[:, n_cls + n_num + c0:]
    return (logits, num_rec, cat0, cat1)
